# baseline (speedup 1.0000x reference)
import jax
from jax import lax
import jax.numpy as jnp
from jax.experimental import pallas as pl
from jax.experimental.pallas import tpu as pltpu
from jax.experimental.pallas import tpu_sc as plsc

B, L = 4096, 50
N = B * L
WORD_VOCAB = 1000000
POS_VOCAB = 1000
WORD_DIM = 64
POS_DIM = 32

NC, NS = 2, 16
NW = NC * NS
PER_W = N // NW
W = 640
N_WIN = PER_W // W
OB = 16
CW = 512


def _transpose_widen(wt_t):
    def body(x_ref, o_ref):
        o_ref[:, 0:WORD_DIM] = x_ref[...].T

    return pl.pallas_call(
        body,
        grid=(pl.cdiv(WORD_VOCAB, CW),),
        in_specs=[pl.BlockSpec((WORD_DIM, CW), lambda i: (0, i))],
        out_specs=pl.BlockSpec((CW, 128), lambda i: (i, 0)),
        out_shape=jax.ShapeDtypeStruct((WORD_VOCAB, 128), jnp.float32),
        compiler_params=pltpu.CompilerParams(
            dimension_semantics=("parallel",)),
    )(wt_t)


def _sc_gather_packed(wt_wide, pos_table, rel_table, widx, pidx, ridx):
    mesh = plsc.VectorSubcoreMesh(core_axis_name="c", subcore_axis_name="s")

    @pl.kernel(
        out_type=jax.ShapeDtypeStruct((N, 128), jnp.float32),
        mesh=mesh,
        scratch_types=[
            pltpu.VMEM((W,), jnp.int32),
            pltpu.VMEM((W,), jnp.int32),
            pltpu.VMEM((W,), jnp.int32),
            pltpu.VMEM((W, 128), jnp.float32),
            pltpu.VMEM((W, POS_DIM), jnp.float32),
            pltpu.VMEM((W, POS_DIM), jnp.float32),
        ],
        compiler_params=pltpu.CompilerParams(use_tc_tiling_on_sc=False),
    )
    def kern(wt_hbm, pt_hbm, rt_hbm, wi_hbm, pi_hbm, ri_hbm, o_hbm,
             wi_v, pi_v, ri_v, wv, pv, rv):
        wid = lax.axis_index("s") * NC + lax.axis_index("c")
        for w in range(N_WIN):
            base = wid * PER_W + w * W
            pltpu.sync_copy(wi_hbm.at[pl.ds(base, W)], wi_v)
            pltpu.sync_copy(pi_hbm.at[pl.ds(base, W)], pi_v)
            pltpu.sync_copy(ri_hbm.at[pl.ds(base, W)], ri_v)
            pltpu.sync_copy(pt_hbm.at[pi_v], pv)
            pltpu.sync_copy(rt_hbm.at[ri_v], rv)
            pltpu.sync_copy(wt_hbm.at[wi_v], wv)
            pltpu.sync_copy(pv, o_hbm.at[pl.ds(base, W), 0:POS_DIM])
            pltpu.sync_copy(rv, o_hbm.at[pl.ds(base, W),
                                         POS_DIM:2 * POS_DIM])
            pltpu.sync_copy(wv.at[:, 0:WORD_DIM],
                            o_hbm.at[pl.ds(base, W), 2 * POS_DIM:128])

    return kern(wt_wide, pos_table, rel_table, widx, pidx, ridx)


def _unpack_outputs(packed):
    def body(x_ref, wo_ref, po_ref, ro_ref):
        x = x_ref[...]
        po_ref[...] = x[:, 0:POS_DIM].reshape(OB, L, POS_DIM)
        ro_ref[...] = x[:, POS_DIM:2 * POS_DIM].reshape(OB, L, POS_DIM)
        wo_ref[...] = x[:, 2 * POS_DIM:128].reshape(OB, L, WORD_DIM)

    return pl.pallas_call(
        body,
        grid=(B // OB,),
        in_specs=[pl.BlockSpec((OB * L, 128), lambda i: (i, 0))],
        out_specs=[
            pl.BlockSpec((OB, L, WORD_DIM), lambda i: (i, 0, 0)),
            pl.BlockSpec((OB, L, POS_DIM), lambda i: (i, 0, 0)),
            pl.BlockSpec((OB, L, POS_DIM), lambda i: (i, 0, 0)),
        ],
        out_shape=(
            jax.ShapeDtypeStruct((B, L, WORD_DIM), jnp.float32),
            jax.ShapeDtypeStruct((B, L, POS_DIM), jnp.float32),
            jax.ShapeDtypeStruct((B, L, POS_DIM), jnp.float32),
        ),
        compiler_params=pltpu.CompilerParams(
            dimension_semantics=("parallel",)),
    )(packed)


@jax.jit
def kernel(word_idxs, pos_idxs, rel_idxs, word_table, pos_table, rel_table):
    wt_wide = _transpose_widen(word_table.T)
    widx = word_idxs.reshape(N)
    pidx = pos_idxs.reshape(N)
    ridx = rel_idxs.reshape(N)
    packed = _sc_gather_packed(wt_wide, pos_table, rel_table,
                               widx, pidx, ridx)
    return _unpack_outputs(packed)


# speedup vs baseline: 1.2064x; 1.2064x over previous
import jax
from jax import lax
import jax.numpy as jnp
from jax.experimental import pallas as pl
from jax.experimental.pallas import tpu as pltpu
from jax.experimental.pallas import tpu_sc as plsc

B, L = 4096, 50
N = B * L
WORD_VOCAB = 1000000
POS_VOCAB = 1000
WORD_DIM = 64
POS_DIM = 32

NC, NS = 2, 16
NW = NC * NS
PER_W = N // NW
W = 640
N_WIN = PER_W // W
CW = 512


def _eye64():
    ii = lax.broadcasted_iota(jnp.int32, (64, 64), 0)
    jj = lax.broadcasted_iota(jnp.int32, (64, 64), 1)
    return (ii == jj).astype(jnp.float32)


def _t64(x, eye):
    return lax.dot_general(x, eye, (((0,), (0,)), ((), ())),
                           preferred_element_type=jnp.float32)


def _transpose_widen(wt_t):
    def body(x_ref, o_ref):
        o_ref[:, 0:WORD_DIM] = _t64(x_ref[...], _eye64())

    return pl.pallas_call(
        body,
        grid=(pl.cdiv(WORD_VOCAB, CW),),
        in_specs=[pl.BlockSpec((WORD_DIM, CW), lambda i: (0, i))],
        out_specs=pl.BlockSpec((CW, 128), lambda i: (i, 0)),
        out_shape=jax.ShapeDtypeStruct((WORD_VOCAB, 128), jnp.float32),
        compiler_params=pltpu.CompilerParams(
            dimension_semantics=("parallel",)),
    )(wt_t)


def _sc_gather_packed(wt_wide, pos_table, rel_table, widx, pidx, ridx):
    mesh = plsc.VectorSubcoreMesh(core_axis_name="c", subcore_axis_name="s")

    @pl.kernel(
        out_type=jax.ShapeDtypeStruct((N, 128), jnp.float32),
        mesh=mesh,
        scratch_types=[
            pltpu.VMEM((W,), jnp.int32),
            pltpu.VMEM((W,), jnp.int32),
            pltpu.VMEM((W,), jnp.int32),
            pltpu.VMEM((W, 128), jnp.float32),
            pltpu.VMEM((W, POS_DIM), jnp.float32),
            pltpu.VMEM((W, POS_DIM), jnp.float32),
        ],
        compiler_params=pltpu.CompilerParams(use_tc_tiling_on_sc=False),
    )
    def kern(wt_hbm, pt_hbm, rt_hbm, wi_hbm, pi_hbm, ri_hbm, o_hbm,
             wi_v, pi_v, ri_v, wv, pv, rv):
        wid = lax.axis_index("s") * NC + lax.axis_index("c")
        for w in range(N_WIN):
            base = wid * PER_W + w * W
            pltpu.sync_copy(wi_hbm.at[pl.ds(base, W)], wi_v)
            pltpu.sync_copy(pi_hbm.at[pl.ds(base, W)], pi_v)
            pltpu.sync_copy(ri_hbm.at[pl.ds(base, W)], ri_v)
            pltpu.sync_copy(pt_hbm.at[pi_v], pv)
            pltpu.sync_copy(rt_hbm.at[ri_v], rv)
            pltpu.sync_copy(wt_hbm.at[wi_v], wv)
            pltpu.sync_copy(pv, o_hbm.at[pl.ds(base, W), 0:POS_DIM])
            pltpu.sync_copy(rv, o_hbm.at[pl.ds(base, W),
                                         POS_DIM:2 * POS_DIM])
            pltpu.sync_copy(wv.at[:, 0:WORD_DIM],
                            o_hbm.at[pl.ds(base, W), 2 * POS_DIM:128])

    return kern(wt_wide, pos_table, rel_table, widx, pidx, ridx)


def _unpack_outputs(packed):
    def body(x_ref, wo_ref, po_ref, ro_ref):
        eye = _eye64()
        for c in range(B // 64):
            t = _t64(x_ref[c * 64:(c + 1) * 64, :], eye)
            sl = slice(c * 64, (c + 1) * 64)
            po_ref[0, :, sl] = t[0:POS_DIM, :]
            ro_ref[0, :, sl] = t[POS_DIM:2 * POS_DIM, :]
            wo_ref[0, :, sl] = t[2 * POS_DIM:128, :]

    return pl.pallas_call(
        body,
        grid=(L,),
        in_specs=[pl.BlockSpec((B, 128), lambda l: (l, 0))],
        out_specs=[
            pl.BlockSpec((1, WORD_DIM, B), lambda l: (l, 0, 0)),
            pl.BlockSpec((1, POS_DIM, B), lambda l: (l, 0, 0)),
            pl.BlockSpec((1, POS_DIM, B), lambda l: (l, 0, 0)),
        ],
        out_shape=(
            jax.ShapeDtypeStruct((L, WORD_DIM, B), jnp.float32),
            jax.ShapeDtypeStruct((L, POS_DIM, B), jnp.float32),
            jax.ShapeDtypeStruct((L, POS_DIM, B), jnp.float32),
        ),
        compiler_params=pltpu.CompilerParams(
            dimension_semantics=("parallel",)),
    )(packed)


@jax.jit
def kernel(word_idxs, pos_idxs, rel_idxs, word_table, pos_table, rel_table):
    wt_wide = _transpose_widen(word_table.T)
    widx = word_idxs.T.reshape(N)
    pidx = pos_idxs.T.reshape(N)
    ridx = rel_idxs.T.reshape(N)
    packed = _sc_gather_packed(wt_wide, pos_table, rel_table,
                               widx, pidx, ridx)
    wo, po, ro = _unpack_outputs(packed)
    return (jnp.transpose(wo, (2, 0, 1)),
            jnp.transpose(po, (2, 0, 1)),
            jnp.transpose(ro, (2, 0, 1)))


# speedup vs baseline: 1.5415x; 1.2778x over previous
import jax
import jax.numpy as jnp
from jax.experimental import pallas as pl
from jax.experimental.pallas import tpu as pltpu
from jax.experimental.pallas import tpu_sc as plsc

B, L = 4096, 50
N = B * L
WORD_VOCAB = 1000000
POS_VOCAB = 1000
WORD_DIM = 64
POS_DIM = 32

W = 128
SC_GRID = N // W
TBLK = 4000
OB = 16


def _widen_word(word_table):
    def body(t_ref, o_ref):
        x = t_ref[...]
        o_ref[:, :WORD_DIM] = x
        o_ref[:, WORD_DIM:] = x

    return pl.pallas_call(
        body,
        grid=(WORD_VOCAB // TBLK,),
        in_specs=[pl.BlockSpec((TBLK, WORD_DIM), lambda i: (i, 0))],
        out_specs=pl.BlockSpec((TBLK, 128), lambda i: (i, 0)),
        out_shape=jax.ShapeDtypeStruct((WORD_VOCAB, 128), jnp.float32),
        compiler_params=pltpu.CompilerParams(
            dimension_semantics=("parallel",)),
    )(word_table)


def _widen_small(pos_table, rel_table):
    def body(p_ref, r_ref, po_ref, ro_ref):
        p = p_ref[...]
        r = r_ref[...]
        for g in range(4):
            po_ref[:, g * POS_DIM:(g + 1) * POS_DIM] = p
            ro_ref[:, g * POS_DIM:(g + 1) * POS_DIM] = r

    return pl.pallas_call(
        body,
        out_shape=(
            jax.ShapeDtypeStruct((POS_VOCAB, 128), jnp.float32),
            jax.ShapeDtypeStruct((POS_VOCAB, 128), jnp.float32),
        ),
    )(pos_table, rel_table)


def _sc_gather_word(wt_wide, widx):
    mesh = plsc.VectorSubcoreMesh(core_axis_name="c", subcore_axis_name="s")

    @pl.kernel(
        out_type=jax.ShapeDtypeStruct((N, 128), jnp.float32),
        mesh=mesh,
        compiler_params=pltpu.CompilerParams(use_tc_tiling_on_sc=False),
    )
    def kern(wt_hbm, wi_hbm, wo_hbm):
        def body(wi_v, wo_v):
            pltpu.sync_copy(wt_hbm.at[wi_v.at[0]], wo_v)

        pltpu.emit_pipeline(
            body,
            grid=(SC_GRID,),
            in_specs=[pl.BlockSpec((1, W), lambda i: (i, 0))],
            out_specs=[pl.BlockSpec((W, 128), lambda i: (i, 0))],
            core_axis_name=("c", "s"),
            dimension_semantics=(pltpu.PARALLEL,),
        )(wi_hbm, wo_hbm)

    return kern(wt_wide, widx)


def _sc_gather_posrel(pt_wide, rt_wide, pidx, ridx):
    mesh = plsc.VectorSubcoreMesh(core_axis_name="c", subcore_axis_name="s")

    @pl.kernel(
        out_type=(
            jax.ShapeDtypeStruct((N, POS_DIM), jnp.float32),
            jax.ShapeDtypeStruct((N, POS_DIM), jnp.float32),
        ),
        mesh=mesh,
        compiler_params=pltpu.CompilerParams(use_tc_tiling_on_sc=False),
    )
    def kern(pt_hbm, rt_hbm, pi_hbm, ri_hbm, po_hbm, ro_hbm):
        def body(pi_v, ri_v, po_v, ro_v):
            pltpu.sync_copy(pt_hbm.at[pi_v.at[0]], po_v)
            pltpu.sync_copy(rt_hbm.at[ri_v.at[0]], ro_v)

        pltpu.emit_pipeline(
            body,
            grid=(SC_GRID,),
            in_specs=[
                pl.BlockSpec((1, W), lambda i: (i, 0)),
                pl.BlockSpec((1, W), lambda i: (i, 0)),
            ],
            out_specs=[
                pl.BlockSpec((W, POS_DIM), lambda i: (i, 0)),
                pl.BlockSpec((W, POS_DIM), lambda i: (i, 0)),
            ],
            core_axis_name=("c", "s"),
            dimension_semantics=(pltpu.PARALLEL,),
        )(pi_hbm, ri_hbm, po_hbm, ro_hbm)

    return kern(pt_wide, rt_wide, pidx, ridx)


def _unpack_outputs(word_wide, pos_wide, rel_wide):
    def body(w_ref, p_ref, r_ref, wo_ref, po_ref, ro_ref):
        wo_ref[...] = w_ref[:, :WORD_DIM].reshape(OB, L, WORD_DIM)
        po_ref[...] = p_ref[:, :POS_DIM].reshape(OB, L, POS_DIM)
        ro_ref[...] = r_ref[:, :POS_DIM].reshape(OB, L, POS_DIM)

    return pl.pallas_call(
        body,
        grid=(B // OB,),
        in_specs=[
            pl.BlockSpec((OB * L, 128), lambda i: (i, 0)),
            pl.BlockSpec((OB * L, 128), lambda i: (i, 0)),
            pl.BlockSpec((OB * L, 128), lambda i: (i, 0)),
        ],
        out_specs=[
            pl.BlockSpec((OB, L, WORD_DIM), lambda i: (i, 0, 0)),
            pl.BlockSpec((OB, L, POS_DIM), lambda i: (i, 0, 0)),
            pl.BlockSpec((OB, L, POS_DIM), lambda i: (i, 0, 0)),
        ],
        out_shape=(
            jax.ShapeDtypeStruct((B, L, WORD_DIM), jnp.float32),
            jax.ShapeDtypeStruct((B, L, POS_DIM), jnp.float32),
            jax.ShapeDtypeStruct((B, L, POS_DIM), jnp.float32),
        ),
        compiler_params=pltpu.CompilerParams(
            dimension_semantics=("parallel",)),
    )(word_wide, pos_wide, rel_wide)


@jax.jit
def kernel(word_idxs, pos_idxs, rel_idxs, word_table, pos_table, rel_table):
    pidx = pos_idxs.reshape(SC_GRID, W)
    ridx = rel_idxs.reshape(SC_GRID, W)
    pos_out, rel_out = _sc_gather_posrel(pos_table, rel_table, pidx, ridx)

    wt_wide = jnp.pad(word_table, ((0, 0), (0, 128 - WORD_DIM)))
    widx = word_idxs.reshape(SC_GRID, W)
    word_wide = _sc_gather_word(wt_wide, widx)

    return (word_wide[:, :WORD_DIM].reshape(B, L, WORD_DIM),
            pos_out.reshape(B, L, POS_DIM),
            rel_out.reshape(B, L, POS_DIM))
